# Initial kernel scaffold; baseline (speedup 1.0000x reference)
import jax
import jax.numpy as jnp
from jax.experimental import pallas as pl


def kernel(node_feature, edge_index):
    raise NotImplementedError("write your pallas kernel here")


# speedup vs baseline: 3.9301x; 3.9301x over previous
import functools

import jax
import jax.numpy as jnp
from jax import lax
from jax.experimental import pallas as pl
from jax.experimental.pallas import tpu as pltpu
from jax.experimental.pallas import tpu_sc as plsc

_STAT_MEDIAN = 0.0
_STAT_MAX = 1.0
_STAT_MIN = -1.0
_SCALE = 2.0 / (_STAT_MAX - _STAT_MIN)

_N_NODES = 10000
_D = 128
_E = 320000

_NC = 2
_NS = 16
_NW = _NC * _NS
_E_PER_W = _E // _NW
_CHUNK = 80
_N_CHUNKS = _E_PER_W // _CHUNK


@functools.partial(
    pl.kernel,
    mesh=plsc.VectorSubcoreMesh(core_axis_name="c", subcore_axis_name="s"),
    out_type=jax.ShapeDtypeStruct((_E, _D), jnp.float32),
    scratch_types=[
        pltpu.VMEM((_E_PER_W,), jnp.int32),
        pltpu.VMEM((_E_PER_W,), jnp.int32),
        pltpu.VMEM((_CHUNK, _D), jnp.float32),
        pltpu.VMEM((_CHUNK, _D), jnp.float32),
        pltpu.SemaphoreType.DMA,
    ],
)
def _edge_diff(table_hbm, src_hbm, dst_hbm, out_hbm, src_v, dst_v, u_v, v_v,
               sem):
    cid = lax.axis_index("c")
    sid = lax.axis_index("s")
    wid = sid * _NC + cid
    base = wid * _E_PER_W

    pltpu.sync_copy(src_hbm.at[pl.ds(base, _E_PER_W)], src_v)
    pltpu.sync_copy(dst_hbm.at[pl.ds(base, _E_PER_W)], dst_v)

    def chunk_body(c, carry):
        off = c * _CHUNK
        cu = pltpu.async_copy(table_hbm.at[src_v.at[pl.ds(off, _CHUNK)]],
                              u_v, sem)
        cv = pltpu.async_copy(table_hbm.at[dst_v.at[pl.ds(off, _CHUNK)]],
                              v_v, sem)
        cu.wait()
        cv.wait()

        def row_body(i, carry2):
            for j in range(_D // 16):
                sl = pl.ds(j * 16, 16)
                u_v[i, sl] = (u_v[i, sl] - v_v[i, sl]) * _SCALE
            return carry2

        lax.fori_loop(0, _CHUNK, row_body, 0)
        pltpu.sync_copy(u_v, out_hbm.at[pl.ds(base + off, _CHUNK)])
        return carry

    lax.fori_loop(0, _N_CHUNKS, chunk_body, 0)


def _norm_body(x_ref, o_ref):
    o_ref[...] = (x_ref[...] - _STAT_MEDIAN) * _SCALE


_norm = pl.pallas_call(
    _norm_body,
    out_shape=jax.ShapeDtypeStruct((_N_NODES, _D), jnp.float32),
    grid=(5,),
    in_specs=[pl.BlockSpec((_N_NODES // 5, _D), lambda i: (i, 0))],
    out_specs=pl.BlockSpec((_N_NODES // 5, _D), lambda i: (i, 0)),
)


def kernel(node_feature, edge_index):
    ei = edge_index.astype(jnp.int32)
    src = ei[0]
    dst = ei[1]
    edge_feature = _edge_diff(node_feature, src, dst)
    norm = _norm(node_feature)
    return (norm, edge_feature)


# speedup vs baseline: 6.8824x; 1.7512x over previous
import functools

import jax
import jax.numpy as jnp
from jax import lax
from jax.experimental import pallas as pl
from jax.experimental.pallas import tpu as pltpu
from jax.experimental.pallas import tpu_sc as plsc

_STAT_MEDIAN = 0.0
_STAT_MAX = 1.0
_STAT_MIN = -1.0
_SCALE = 2.0 / (_STAT_MAX - _STAT_MIN)

_N_NODES = 10000
_D = 128
_E = 320000

_NC = 2
_NS = 16
_NW = _NC * _NS
_E_PER_W = _E // _NW
_CHUNK = 80
_N_CHUNKS = _E_PER_W // _CHUNK


@functools.partial(
    pl.kernel,
    mesh=plsc.VectorSubcoreMesh(core_axis_name="c", subcore_axis_name="s"),
    out_type=jax.ShapeDtypeStruct((_E, _D), jnp.float32),
    scratch_types=[
        pltpu.VMEM((_E_PER_W,), jnp.int32),
        pltpu.VMEM((_E_PER_W,), jnp.int32),
        pltpu.VMEM((_CHUNK, _D), jnp.float32),
        pltpu.VMEM((_CHUNK, _D), jnp.float32),
        pltpu.VMEM((_CHUNK, _D), jnp.float32),
        pltpu.VMEM((_CHUNK, _D), jnp.float32),
        pltpu.VMEM((_CHUNK, _D), jnp.float32),
        pltpu.VMEM((_CHUNK, _D), jnp.float32),
        pltpu.SemaphoreType.DMA,
        pltpu.SemaphoreType.DMA,
        pltpu.SemaphoreType.DMA,
        pltpu.SemaphoreType.DMA,
    ],
)
def _edge_diff(table_hbm, src_hbm, dst_hbm, out_hbm, src_v, dst_v,
               u0, v0, o0, u1, v1, o1, g0, g1, w0, w1):
    cid = lax.axis_index("c")
    sid = lax.axis_index("s")
    wid = sid * _NC + cid
    base = wid * _E_PER_W

    pltpu.sync_copy(src_hbm.at[pl.ds(base, _E_PER_W)], src_v)
    pltpu.sync_copy(dst_hbm.at[pl.ds(base, _E_PER_W)], dst_v)

    bufs = ((u0, v0, o0, g0, w0), (u1, v1, o1, g1, w1))

    def start_gather(c, b):
        ub, vb, _, g, _w = bufs[b]
        off = c * _CHUNK
        pltpu.async_copy(table_hbm.at[src_v.at[pl.ds(off, _CHUNK)]], ub, g)
        pltpu.async_copy(table_hbm.at[dst_v.at[pl.ds(off, _CHUNK)]], vb, g)

    def wait_gather(c, b):
        ub, vb, _, g, _w = bufs[b]
        off = c * _CHUNK
        pltpu.make_async_copy(
            table_hbm.at[src_v.at[pl.ds(off, _CHUNK)]], ub, g).wait()
        pltpu.make_async_copy(
            table_hbm.at[dst_v.at[pl.ds(off, _CHUNK)]], vb, g).wait()

    def start_write(c, b):
        _u, _v, ob, _g, w = bufs[b]
        pltpu.async_copy(ob, out_hbm.at[pl.ds(base + c * _CHUNK, _CHUNK)], w)

    def wait_write(b):
        _u, _v, ob, _g, w = bufs[b]
        pltpu.make_async_copy(ob, out_hbm.at[pl.ds(base, _CHUNK)], w).wait()

    def compute(b):
        ub, vb, ob, _g, _w = bufs[b]

        def row(i, carry):
            for j in range(_D // 16):
                sl = pl.ds(j * 16, 16)
                ob[i, sl] = (ub[i, sl] - vb[i, sl]) * _SCALE
            return carry

        lax.fori_loop(0, _CHUNK, row, 0)

    start_gather(0, 0)

    def body(i, carry):
        c = 2 * i

        def stage(cc, b):
            start_gather(cc + 1, 1 - b)
            wait_gather(cc, b)
            pl.when(i >= 1)(lambda: wait_write(b))
            compute(b)
            start_write(cc, b)

        stage(c, 0)
        stage(c + 1, 1)
        return carry

    lax.fori_loop(0, (_N_CHUNKS - 1) // 2, body, 0)

    wait_gather(_N_CHUNKS - 1, 0)
    wait_write(0)
    compute(0)
    start_write(_N_CHUNKS - 1, 0)
    wait_write(0)
    wait_write(1)


def _norm_body(x_ref, o_ref):
    o_ref[...] = (x_ref[...] - _STAT_MEDIAN) * _SCALE


_norm = pl.pallas_call(
    _norm_body,
    out_shape=jax.ShapeDtypeStruct((_N_NODES, _D), jnp.float32),
    grid=(5,),
    in_specs=[pl.BlockSpec((_N_NODES // 5, _D), lambda i: (i, 0))],
    out_specs=pl.BlockSpec((_N_NODES // 5, _D), lambda i: (i, 0)),
)


def kernel(node_feature, edge_index):
    ei = edge_index.astype(jnp.int32)
    src = ei[0]
    dst = ei[1]
    edge_feature = _edge_diff(node_feature, src, dst)
    norm = _norm(node_feature)
    return (norm, edge_feature)


# speedup vs baseline: 7.1453x; 1.0382x over previous
import functools

import jax
import jax.numpy as jnp
from jax import lax
from jax.experimental import pallas as pl
from jax.experimental.pallas import tpu as pltpu
from jax.experimental.pallas import tpu_sc as plsc

_STAT_MEDIAN = 0.0
_STAT_MAX = 1.0
_STAT_MIN = -1.0
_SCALE = 2.0 / (_STAT_MAX - _STAT_MIN)

_N_NODES = 10000
_D = 128
_E = 320000

_NC = 2
_NS = 16
_NW = _NC * _NS
_E_PER_W = _E // _NW
_CHUNK = 80
_N_CHUNKS = _E_PER_W // _CHUNK
_NBUF = 4


@functools.partial(
    pl.kernel,
    mesh=plsc.VectorSubcoreMesh(core_axis_name="c", subcore_axis_name="s"),
    out_type=jax.ShapeDtypeStruct((_E, _D), jnp.float32),
    scratch_types=(
        [pltpu.VMEM((_E_PER_W,), jnp.int32)] * 2
        + [pltpu.VMEM((_CHUNK, _D), jnp.float32)] * (2 * _NBUF)
        + [pltpu.SemaphoreType.DMA] * (2 * _NBUF)
    ),
)
def _edge_diff(table_hbm, src_hbm, dst_hbm, out_hbm, src_v, dst_v,
               u0, v0, u1, v1, u2, v2, u3, v3,
               g0, g1, g2, g3, w0, w1, w2, w3):
    cid = lax.axis_index("c")
    sid = lax.axis_index("s")
    wid = sid * _NC + cid
    base = wid * _E_PER_W

    pltpu.sync_copy(src_hbm.at[pl.ds(base, _E_PER_W)], src_v)
    pltpu.sync_copy(dst_hbm.at[pl.ds(base, _E_PER_W)], dst_v)

    bufs = ((u0, v0, g0, w0), (u1, v1, g1, w1),
            (u2, v2, g2, w2), (u3, v3, g3, w3))

    def start_gather(c, k):
        ub, vb, g, _w = bufs[k]
        off = c * _CHUNK
        pltpu.async_copy(table_hbm.at[src_v.at[pl.ds(off, _CHUNK)]], ub, g)
        pltpu.async_copy(table_hbm.at[dst_v.at[pl.ds(off, _CHUNK)]], vb, g)

    def wait_gather(c, k):
        ub, vb, g, _w = bufs[k]
        off = c * _CHUNK
        pltpu.make_async_copy(
            table_hbm.at[src_v.at[pl.ds(off, _CHUNK)]], ub, g).wait()
        pltpu.make_async_copy(
            table_hbm.at[dst_v.at[pl.ds(off, _CHUNK)]], vb, g).wait()

    def start_write(c, k):
        ub, _v, _g, w = bufs[k]
        pltpu.async_copy(ub, out_hbm.at[pl.ds(base + c * _CHUNK, _CHUNK)], w)

    def wait_write(k):
        ub, _v, _g, w = bufs[k]
        pltpu.make_async_copy(ub, out_hbm.at[pl.ds(base, _CHUNK)], w).wait()

    def compute(k):
        ub, vb, _g, _w = bufs[k]

        def row(i, carry):
            for j in range(_D // 16):
                sl = pl.ds(j * 16, 16)
                ub[i, sl] = (ub[i, sl] - vb[i, sl]) * _SCALE
            return carry

        lax.fori_loop(0, _CHUNK, row, 0)

    start_gather(0, 0)
    start_gather(1, 1)

    def body(i, carry):
        for k in range(_NBUF):
            cc = _NBUF * i + k
            kn = (k + 2) % _NBUF
            pl.when(cc >= 2)(lambda: wait_write(kn))
            pl.when(cc + 2 < _N_CHUNKS)(lambda: start_gather(cc + 2, kn))
            wait_gather(cc, k)
            compute(k)
            start_write(cc, k)
        return carry

    lax.fori_loop(0, (_N_CHUNKS - 1) // _NBUF, body, 0)

    last = _N_CHUNKS - 1
    wait_gather(last, last % _NBUF)
    compute(last % _NBUF)
    start_write(last, last % _NBUF)
    wait_write(2)
    wait_write(3)
    wait_write(0)


def _norm_body(x_ref, o_ref):
    o_ref[...] = (x_ref[...] - _STAT_MEDIAN) * _SCALE


_norm = pl.pallas_call(
    _norm_body,
    out_shape=jax.ShapeDtypeStruct((_N_NODES, _D), jnp.float32),
    grid=(5,),
    in_specs=[pl.BlockSpec((_N_NODES // 5, _D), lambda i: (i, 0))],
    out_specs=pl.BlockSpec((_N_NODES // 5, _D), lambda i: (i, 0)),
)


def kernel(node_feature, edge_index):
    ei = edge_index.astype(jnp.int32)
    src = ei[0]
    dst = ei[1]
    edge_feature = _edge_diff(node_feature, src, dst)
    norm = _norm(node_feature)
    return (norm, edge_feature)
